# baseline (speedup 1.0000x reference)
import functools

import jax
import jax.numpy as jnp
from jax import lax
from jax.experimental import pallas as pl
from jax.experimental.pallas import tpu as pltpu
from jax.experimental.pallas import tpu_sc as plsc

N = 10000
E = 320000
D = 128
EMB = 10

NC = 2
NS = 16
L = 16
NW = NC * NS
EPW = E // NW

_MESH = plsc.VectorSubcoreMesh(
    core_axis_name="c", subcore_axis_name="s", num_cores=NC, num_subcores=NS)


def _worker_id():
    return lax.axis_index("s") * NC + lax.axis_index("c")


def _zero_vmem(ref, n):
    def body(i, _):
        ref[pl.ds(i * L, L)] = jnp.zeros((L,), jnp.float32)
        return 0
    lax.fori_loop(0, n // L, body, 0, unroll=8)


@functools.partial(
    pl.kernel,
    out_type=jax.ShapeDtypeStruct((NW * N,), jnp.float32),
    mesh=_MESH,
    compiler_params=pltpu.CompilerParams(needs_layout_passes=False),
    scratch_types=[
        pltpu.VMEM((EPW,), jnp.int32),
        pltpu.VMEM((EPW,), jnp.int32),
        pltpu.VMEM((N,), jnp.float32),
    ],
)
def _deg_pass(row_hbm, col_hbm, out_hbm, row_v, col_v, acc_v):
    wid = _worker_id()
    base = wid * EPW
    pltpu.sync_copy(row_hbm.at[pl.ds(base, EPW)], row_v)
    pltpu.sync_copy(col_hbm.at[pl.ds(base, EPW)], col_v)
    _zero_vmem(acc_v, N)
    ones = jnp.ones((L,), jnp.float32)

    def body(i, _):
        r = row_v[pl.ds(i * L, L)]
        c = col_v[pl.ds(i * L, L)]
        plsc.addupdate_scatter(acc_v, [r], ones, mask=r != c)
        return 0

    lax.fori_loop(0, EPW // L, body, 0, unroll=8)
    pltpu.sync_copy(acc_v, out_hbm.at[pl.ds(wid * N, N)])


@functools.partial(
    pl.kernel,
    out_type=jax.ShapeDtypeStruct((NW * N,), jnp.float32),
    mesh=_MESH,
    compiler_params=pltpu.CompilerParams(needs_layout_passes=False),
    scratch_types=[
        pltpu.VMEM((EPW,), jnp.int32),
        pltpu.VMEM((EPW,), jnp.int32),
        pltpu.VMEM((N,), jnp.float32),
        pltpu.VMEM((N,), jnp.float32),
    ],
)
def _gather_pass(row_hbm, col_hbm, u_hbm, out_hbm, row_v, col_v, u_v, acc_v):
    wid = _worker_id()
    base = wid * EPW
    pltpu.sync_copy(row_hbm.at[pl.ds(base, EPW)], row_v)
    pltpu.sync_copy(col_hbm.at[pl.ds(base, EPW)], col_v)
    pltpu.sync_copy(u_hbm, u_v)
    _zero_vmem(acc_v, N)

    def body(i, _):
        r = row_v[pl.ds(i * L, L)]
        c = col_v[pl.ds(i * L, L)]
        vals = plsc.load_gather(u_v, [r])
        plsc.addupdate_scatter(acc_v, [c], vals, mask=r != c)
        return 0

    lax.fori_loop(0, EPW // L, body, 0, unroll=8)
    pltpu.sync_copy(acc_v, out_hbm.at[pl.ds(wid * N, N)])


def _prep_body(deg_ref, x_ref, w01_ref, bm_ref, u_ref, ad_ref):
    deg = jnp.sum(deg_ref[...], axis=0)
    safe = jnp.where(deg > 0, deg, 1.0)
    dinv = jnp.where(deg > 0, lax.rsqrt(safe), 0.0)
    xv = x_ref[...]
    xb = xv.astype(jnp.bfloat16).astype(jnp.float32)
    t = jnp.sum(xb * w01_ref[0:1, :], axis=1)
    s = jnp.sum(xv * w01_ref[1:2, :], axis=1)
    u_ref[...] = dinv * s
    ad_ref[...] = jnp.stack([t + bm_ref[0, 0], dinv], axis=0)


def _prep(deg_parts, x, w01, bm):
    return pl.pallas_call(
        _prep_body,
        out_shape=(jax.ShapeDtypeStruct((N,), jnp.float32),
                   jax.ShapeDtypeStruct((2, N), jnp.float32)),
    )(deg_parts, x, w01, bm)


BR = 80
NSTR = 5
GBR = BR * NSTR
NSTEP = N // GBR


def _dense_body(w0_ref, w1_ref, w2_ref, w3_ref, w4_ref, ad_ref, g_ref,
                db_ref, fwt_ref, fb_ref, out_ref, h0_s, emb_s):
    i = pl.program_id(0)

    @pl.when(i == 0)
    def _init():
        g = jnp.sum(g_ref[...], axis=0, keepdims=True)
        h0_s[...] = ad_ref[0:1, :] - ad_ref[1:2, :] * g
        emb_s[...] = jnp.zeros_like(emb_s)

    h0 = h0_s[...]
    acc = emb_s[...]
    for k, w_ref in enumerate((w0_ref, w1_ref, w2_ref, w3_ref, w4_ref)):
        acc += w_ref[0, 0:1, 0:EMB] + h0[0:1, 0:EMB]
    emb_s[...] = acc

    @pl.when(i == NSTEP - 1)
    def _fin():
        out_ref[...] = emb_s[...] + fb_ref[...]


def _dense(dense_W, ad, g_parts, db_col, fwt, fb_row):
    dense_W = dense_W.reshape(3125, 25, 1280)
    wspec = [pl.BlockSpec((25, 25, 1280), functools.partial(
        lambda k, i: (NSTR * i + k, 0, 0), k)) for k in range(NSTR)]
    return pl.pallas_call(
        _dense_body,
        grid=(NSTEP,),
        in_specs=wspec + [
            pl.BlockSpec((2, N), lambda i: (0, 0)),
            pl.BlockSpec((NW, N), lambda i: (0, 0)),
            pl.BlockSpec((GBR, 1), lambda i: (i, 0)),
            pl.BlockSpec((GBR, EMB), lambda i: (i, 0)),
            pl.BlockSpec((1, EMB), lambda i: (0, 0)),
        ],
        out_specs=pl.BlockSpec((1, EMB), lambda i: (0, 0)),
        out_shape=jax.ShapeDtypeStruct((1, EMB), jnp.float32),
        scratch_shapes=[
            pltpu.VMEM((1, N), jnp.float32),
            pltpu.VMEM((1, EMB), jnp.float32),
        ],
    )(dense_W, dense_W, dense_W, dense_W, dense_W, ad, g_parts, db_col,
      fwt, fb_row)


def kernel(x, edge_index, W0, W1, cheb_b, dense_W, dense_b, final_W, final_b):
    row1d = edge_index[0]
    col1d = edge_index[1]
    deg_parts = _deg_pass(row1d, col1d).reshape(NW, N)
    W0r = W0.astype(jnp.bfloat16).astype(jnp.float32)
    W1r = W1.astype(jnp.bfloat16).astype(jnp.float32)
    w01 = jnp.stack([jnp.mean(W0r, axis=1), jnp.mean(W1r, axis=1)])
    bm = jnp.mean(cheb_b).reshape(1, 1)
    u, ad = _prep(deg_parts, x, w01, bm)
    g_parts = _gather_pass(row1d, col1d, u).reshape(NW, N)
    emb = _dense(dense_W, ad, g_parts,
                 dense_b.reshape(N, 1), final_W.T,
                 final_b.reshape(1, EMB))
    return emb.reshape(EMB)


# speedup vs baseline: 3.7440x; 3.7440x over previous
import functools

import jax
import jax.numpy as jnp
from jax import lax
from jax.experimental import pallas as pl
from jax.experimental.pallas import tpu as pltpu
from jax.experimental.pallas import tpu_sc as plsc

N = 10000
E = 320000
D = 128
EMB = 10

NC = 2
NS = 16
L = 16
NW = NC * NS
EPW = E // NW

_MESH = plsc.VectorSubcoreMesh(
    core_axis_name="c", subcore_axis_name="s", num_cores=NC, num_subcores=NS)


def _worker_id():
    return lax.axis_index("s") * NC + lax.axis_index("c")


def _zero_vmem(ref, n):
    def body(i, _):
        ref[pl.ds(i * L, L)] = jnp.zeros((L,), jnp.float32)
        return 0
    lax.fori_loop(0, n // L, body, 0, unroll=8)


@functools.partial(
    pl.kernel,
    out_type=jax.ShapeDtypeStruct((NW * N,), jnp.float32),
    mesh=_MESH,
    compiler_params=pltpu.CompilerParams(needs_layout_passes=False),
    scratch_types=[
        pltpu.VMEM((EPW,), jnp.int32),
        pltpu.VMEM((EPW,), jnp.int32),
        pltpu.VMEM((N,), jnp.float32),
    ],
)
def _deg_pass(row_hbm, col_hbm, out_hbm, row_v, col_v, acc_v):
    wid = _worker_id()
    base = wid * EPW
    pltpu.sync_copy(row_hbm.at[pl.ds(base, EPW)], row_v)
    pltpu.sync_copy(col_hbm.at[pl.ds(base, EPW)], col_v)
    _zero_vmem(acc_v, N)
    ones = jnp.ones((L,), jnp.float32)

    def body(i, _):
        r = row_v[pl.ds(i * L, L)]
        c = col_v[pl.ds(i * L, L)]
        plsc.addupdate_scatter(acc_v, [r], ones, mask=r != c)
        return 0

    lax.fori_loop(0, EPW // L, body, 0, unroll=16)
    pltpu.sync_copy(acc_v, out_hbm.at[pl.ds(wid * N, N)])


@functools.partial(
    pl.kernel,
    out_type=jax.ShapeDtypeStruct((NW * N,), jnp.float32),
    mesh=_MESH,
    compiler_params=pltpu.CompilerParams(needs_layout_passes=False),
    scratch_types=[
        pltpu.VMEM((EPW,), jnp.int32),
        pltpu.VMEM((EPW,), jnp.int32),
        pltpu.VMEM((N,), jnp.float32),
        pltpu.VMEM((N,), jnp.float32),
    ],
)
def _gather_pass(row_hbm, col_hbm, u_hbm, out_hbm, row_v, col_v, u_v, acc_v):
    wid = _worker_id()
    base = wid * EPW
    pltpu.sync_copy(row_hbm.at[pl.ds(base, EPW)], row_v)
    pltpu.sync_copy(col_hbm.at[pl.ds(base, EPW)], col_v)
    pltpu.sync_copy(u_hbm, u_v)
    _zero_vmem(acc_v, N)

    def body(i, _):
        r = row_v[pl.ds(i * L, L)]
        c = col_v[pl.ds(i * L, L)]
        vals = plsc.load_gather(u_v, [r])
        plsc.addupdate_scatter(acc_v, [c], vals, mask=r != c)
        return 0

    lax.fori_loop(0, EPW // L, body, 0, unroll=16)
    pltpu.sync_copy(acc_v, out_hbm.at[pl.ds(wid * N, N)])


def _prep_st_body(x_ref, w01_ref, bm_ref, sa_ref):
    xv = x_ref[...]
    xb = xv.astype(jnp.bfloat16).astype(jnp.float32)
    t = jnp.sum(xb * w01_ref[0:1, :], axis=1)
    s = jnp.sum(xv * w01_ref[1:2, :], axis=1)
    sa_ref[...] = jnp.stack([t + bm_ref[0, 0], s], axis=0)


def _prep_st(x, w01, bm):
    return pl.pallas_call(
        _prep_st_body,
        out_shape=jax.ShapeDtypeStruct((2, N), jnp.float32),
    )(x, w01, bm)


def _prep_u_body(deg_ref, sa_ref, u_ref, ad_ref):
    deg = jnp.sum(deg_ref[...], axis=0)
    safe = jnp.where(deg > 0, deg, 1.0)
    dinv = jnp.where(deg > 0, lax.rsqrt(safe), 0.0)
    u_ref[...] = dinv * sa_ref[1, :]
    ad_ref[...] = jnp.stack([sa_ref[0, :], dinv], axis=0)


def _prep_u(deg_parts, sa):
    return pl.pallas_call(
        _prep_u_body,
        out_shape=(jax.ShapeDtypeStruct((N,), jnp.float32),
                   jax.ShapeDtypeStruct((2, N), jnp.float32)),
    )(deg_parts, sa)


BR = 80
NSTR = 5
GBR = BR * NSTR
NSTEP = N // GBR


def _dense_body(w0_ref, w1_ref, w2_ref, w3_ref, w4_ref, ad_ref, g_ref,
                db_ref, fwt_ref, fb_ref, out_ref, h0_s, emb_s):
    i = pl.program_id(0)

    @pl.when(i == 0)
    def _init():
        g = jnp.sum(g_ref[...], axis=0, keepdims=True)
        h0_s[...] = ad_ref[0:1, :] - ad_ref[1:2, :] * g
        emb_s[...] = jnp.zeros_like(emb_s)

    h0 = h0_s[...]
    acc = emb_s[...]
    for k, w_ref in enumerate((w0_ref, w1_ref, w2_ref, w3_ref, w4_ref)):
        hblk = lax.dot_general(w_ref[...], h0,
                               (((1,), (1,)), ((), ())))
        hrow = hblk[:, 0] + db_ref[pl.ds(k * BR, BR), 0]
        acc += lax.dot_general(hrow[None, :],
                               fwt_ref[pl.ds(k * BR, BR), :],
                               (((1,), (0,)), ((), ())))
    emb_s[...] = acc

    @pl.when(i == NSTEP - 1)
    def _fin():
        out_ref[...] = emb_s[...] + fb_ref[...]


def _dense(dense_W, ad, g_parts, db_col, fwt, fb_row):
    wspec = [pl.BlockSpec((BR, N), functools.partial(
        lambda k, i: (NSTR * i + k, 0), k)) for k in range(NSTR)]
    return pl.pallas_call(
        _dense_body,
        grid=(NSTEP,),
        in_specs=wspec + [
            pl.BlockSpec((2, N), lambda i: (0, 0)),
            pl.BlockSpec((NW, N), lambda i: (0, 0)),
            pl.BlockSpec((GBR, 1), lambda i: (i, 0)),
            pl.BlockSpec((GBR, EMB), lambda i: (i, 0)),
            pl.BlockSpec((1, EMB), lambda i: (0, 0)),
        ],
        out_specs=pl.BlockSpec((1, EMB), lambda i: (0, 0)),
        out_shape=jax.ShapeDtypeStruct((1, EMB), jnp.float32),
        scratch_shapes=[
            pltpu.VMEM((1, N), jnp.float32),
            pltpu.VMEM((1, EMB), jnp.float32),
        ],
    )(dense_W, dense_W, dense_W, dense_W, dense_W, ad, g_parts, db_col,
      fwt, fb_row)


def kernel(x, edge_index, W0, W1, cheb_b, dense_W, dense_b, final_W, final_b):
    row1d = edge_index[0]
    col1d = edge_index[1]
    W0r = W0.astype(jnp.bfloat16).astype(jnp.float32)
    W1r = W1.astype(jnp.bfloat16).astype(jnp.float32)
    w01 = jnp.stack([jnp.mean(W0r, axis=1), jnp.mean(W1r, axis=1)])
    bm = jnp.mean(cheb_b).reshape(1, 1)
    sa = _prep_st(x, w01, bm)
    deg_parts = _deg_pass(row1d, col1d).reshape(NW, N)
    u, ad = _prep_u(deg_parts, sa)
    g_parts = _gather_pass(row1d, col1d, u).reshape(NW, N)
    emb = _dense(dense_W, ad, g_parts,
                 dense_b.reshape(N, 1), final_W.T,
                 final_b.reshape(1, EMB))
    return emb.reshape(EMB)
